# baseline (speedup 1.0000x reference)
import jax
import jax.numpy as jnp
from jax import lax
from jax.experimental import pallas as pl
from jax.experimental.pallas import tpu as pltpu
from jax.experimental.pallas import tpu_sc as plsc

B = 16384
S = 200
V = 100000
C = 16

NC = 2
NS = 16
NW = NC * NS
BPW = B // NW
GROUP = 16
TOK = GROUP * S
IDXW = 128
NGATH = TOK // IDXW
NROWS = B * S // IDXW
RPW = NROWS // NW
STG = NGATH + 7
NGROUP = BPW // GROUP
NACC = 8


def _sc_body(idx_hbm, emb_hbm, out_hbm, idx_v, rows_v, acc_v, gsem0, gsem1):
    wid = lax.axis_index("s") * NC + lax.axis_index("c")
    row0 = wid * RPW

    def stage_and_fire(buf, g, sem):
        lead = lax.rem(g, 8)
        base = row0 + g * NGATH - lead
        pltpu.sync_copy(idx_hbm.at[pl.ds(base, STG)], idx_v.at[buf])
        for c in range(NGATH):
            pltpu.async_copy(
                emb_hbm.at[idx_v.at[buf, lead + c]],
                rows_v.at[buf, pl.ds(c * IDXW, IDXW)],
                sem,
            )

    def drain(buf, sem):
        pltpu.make_async_copy(emb_hbm.at[pl.ds(0, TOK)], rows_v.at[buf],
                              sem).wait()

    def accumulate(buf, g):
        def sample_body(i, _):
            base = i * S
            a = [jnp.zeros((2, 16), jnp.bfloat16) for _ in range(NACC)]
            for j in range(S // 2):
                a[j % NACC] = a[j % NACC] + rows_v[buf,
                                                   pl.ds(base + 2 * j, 2), :]
            a = [a[0] + a[1], a[2] + a[3], a[4] + a[5], a[6] + a[7]]
            acc_v[pl.ds(2 * i, 2), :] = (a[0] + a[1]) + (a[2] + a[3])
            return 0

        lax.fori_loop(0, GROUP, sample_body, 0)
        pltpu.sync_copy(
            acc_v,
            out_hbm.at[pl.ds(2 * (wid * BPW + g * GROUP), 2 * GROUP)])

    stage_and_fire(0, 0, gsem0)

    def pair_body(gg, _):
        g0 = 2 * gg
        stage_and_fire(1, g0 + 1, gsem1)
        drain(0, gsem0)
        accumulate(0, g0)

        @pl.when(gg != NGROUP // 2 - 1)
        def _():
            stage_and_fire(0, g0 + 2, gsem0)

        drain(1, gsem1)
        accumulate(1, g0 + 1)
        return 0

    lax.fori_loop(0, NGROUP // 2, pair_body, 0)


_sc_pool = pl.kernel(
    _sc_body,
    out_type=jax.ShapeDtypeStruct((2 * B, C), jnp.bfloat16),
    mesh=plsc.VectorSubcoreMesh(
        core_axis_name="c", subcore_axis_name="s", num_cores=NC,
        num_subcores=NS),
    scratch_types=[
        pltpu.VMEM((2, STG, IDXW), jnp.int32),
        pltpu.VMEM((2, TOK, C), jnp.bfloat16),
        pltpu.VMEM((2 * GROUP, C), jnp.bfloat16),
        pltpu.SemaphoreType.DMA,
        pltpu.SemaphoreType.DMA,
    ],
    compiler_params=pltpu.CompilerParams(use_tc_tiling_on_sc=False,
                                         needs_layout_passes=False),
)


def _tc_body(x_ref, b_ref, o_ref):
    x = x_ref[...].astype(jnp.float32)
    x3 = x.reshape(_TCBLK, 2, C)
    logits = x3[:, 0, :] + x3[:, 1, :] + b_ref[0:1, :]
    m = jnp.max(logits, axis=-1, keepdims=True)
    e = jnp.exp(logits - m)
    lse = jnp.log(jnp.sum(e, axis=-1, keepdims=True))
    o_ref[...] = (logits - m) - lse


_TCBLK = 2048
_tc_finish = pl.pallas_call(
    _tc_body,
    out_shape=jax.ShapeDtypeStruct((B, C), jnp.float32),
    grid=(B // _TCBLK,),
    in_specs=[pl.BlockSpec((2 * _TCBLK, C), lambda i: (i, 0)),
              pl.BlockSpec((8, C), lambda i: (0, 0))],
    out_specs=pl.BlockSpec((_TCBLK, C), lambda i: (i, 0)),
)


def kernel(text, emb, bias):
    part = _sc_pool(text.reshape(NROWS, IDXW), emb.astype(jnp.bfloat16))
    return _tc_finish(part, jnp.broadcast_to(bias.reshape(1, C), (8, C)))
